# baseline (speedup 1.0000x reference)
import jax
import jax.numpy as jnp
from jax.experimental import pallas as pl
from jax.experimental.pallas import tpu as pltpu

D = 256
K = 8192
N = 8192
BETA = 0.25
TILE = 128
STEPS = N // TILE


def _vq_kernel(x_ref, xsq_ref, esq_ref, emb_ref,
               dist_ref, enc_ref, quant_ref, ids_ref, loss_ref, perp_ref,
               counts_ref, loss_acc_ref):
    i = pl.program_id(0)
    x = x_ref[...]
    emb = emb_ref[...]
    cross = jax.lax.dot_general(x, emb, (((1,), (0,)), ((), ())),
                                preferred_element_type=jnp.float32)
    dist = xsq_ref[...] + esq_ref[...] - 2.0 * cross
    dist_ref[...] = dist

    m = jnp.min(dist, axis=1, keepdims=True)
    col = jax.lax.broadcasted_iota(jnp.int32, (TILE, K), 1)
    ids = jnp.min(jnp.where(dist == m, col, K), axis=1).astype(jnp.int32)
    enc = (col == ids[:, None]).astype(jnp.float32)
    enc_ref[...] = enc
    ids_ref[...] = ids.reshape(1, 1, TILE)

    quant = jax.lax.dot_general(enc, emb, (((1,), (1,)), ((), ())),
                                precision=jax.lax.Precision.HIGHEST,
                                preferred_element_type=jnp.float32)
    quant_ref[...] = quant

    part_counts = jnp.sum(enc, axis=0, keepdims=True)
    diff = x - quant
    part_loss = jnp.sum(diff * diff)

    @pl.when(i == 0)
    def _init():
        counts_ref[...] = part_counts
        loss_acc_ref[0, 0] = part_loss

    @pl.when(i > 0)
    def _acc():
        counts_ref[...] += part_counts
        loss_acc_ref[0, 0] += part_loss

    @pl.when(i == STEPS - 1)
    def _fin():
        loss_val = BETA * loss_acc_ref[0, 0] / (N * D)
        loss_ref[...] = jnp.full((1, 1), loss_val, jnp.float32)
        p = counts_ref[...] * (1.0 / N)
        ent = jnp.sum(p * jnp.log(p + 1e-10))
        perp_ref[...] = jnp.full((1, 1), jnp.exp(-ent), jnp.float32)


def kernel(x, embedding):
    B, _, H, W = x.shape
    xp = jnp.transpose(x, (0, 2, 3, 1))
    x_flat = xp.reshape(-1, D)
    xsq = jnp.sum(x_flat ** 2, axis=1, keepdims=True)
    esq = jnp.sum(embedding ** 2, axis=0, keepdims=True)

    grid = (STEPS,)
    dist_out, enc_out, quant_out, ids_out, loss_out, perp_out = pl.pallas_call(
        _vq_kernel,
        grid=grid,
        in_specs=[
            pl.BlockSpec((TILE, D), lambda i: (i, 0)),
            pl.BlockSpec((TILE, 1), lambda i: (i, 0)),
            pl.BlockSpec((1, K), lambda i: (0, 0)),
            pl.BlockSpec((D, K), lambda i: (0, 0)),
        ],
        out_specs=[
            pl.BlockSpec((TILE, K), lambda i: (i, 0)),
            pl.BlockSpec((TILE, K), lambda i: (i, 0)),
            pl.BlockSpec((TILE, D), lambda i: (i, 0)),
            pl.BlockSpec((1, 1, TILE), lambda i: (i, 0, 0)),
            pl.BlockSpec((1, 1), lambda i: (0, 0)),
            pl.BlockSpec((1, 1), lambda i: (0, 0)),
        ],
        out_shape=[
            jax.ShapeDtypeStruct((N, K), jnp.float32),
            jax.ShapeDtypeStruct((N, K), jnp.float32),
            jax.ShapeDtypeStruct((N, D), jnp.float32),
            jax.ShapeDtypeStruct((STEPS, 1, TILE), jnp.int32),
            jax.ShapeDtypeStruct((1, 1), jnp.float32),
            jax.ShapeDtypeStruct((1, 1), jnp.float32),
        ],
        scratch_shapes=[
            pltpu.VMEM((1, K), jnp.float32),
            pltpu.SMEM((1, 1), jnp.float32),
        ],
        compiler_params=pltpu.CompilerParams(
            dimension_semantics=("arbitrary",),
        ),
    )(x_flat, xsq, esq, embedding)

    out = jnp.transpose(quant_out.reshape(B, H, W, D), (0, 3, 1, 2))
    loss = loss_out[0, 0]
    perplexity = perp_out[0, 0]
    ids_grid = ids_out.reshape(B, H, W)
    return (out, loss, perplexity, enc_out, ids_grid, dist_out)


# speedup vs baseline: 2.0293x; 2.0293x over previous
import jax
import jax.numpy as jnp
from jax.experimental import pallas as pl
from jax.experimental.pallas import tpu as pltpu

D = 256
K = 8192
N = 8192
BETA = 0.25
TILE = 128
STEPS = N // TILE


def _vq_kernel(x_ref, xsq_ref, esq_ref, emb_ref,
               dist_ref, enc_ref, quant_ref, ids_ref, loss_ref, perp_ref,
               counts_ref, loss_acc_ref):
    i = pl.program_id(0)
    x = x_ref[...]
    emb = emb_ref[...]
    cross = jax.lax.dot_general(x, emb, (((1,), (0,)), ((), ())),
                                preferred_element_type=jnp.float32)
    dist = xsq_ref[...] + esq_ref[...] - 2.0 * cross
    dist_ref[...] = dist

    m = jnp.min(dist, axis=1, keepdims=True)
    col = jax.lax.broadcasted_iota(jnp.int32, (TILE, K), 1)
    ids = jnp.min(jnp.where(dist == m, col, K), axis=1).astype(jnp.int32)
    enc = (col == ids[:, None]).astype(jnp.float32)
    enc_ref[...] = enc
    ids_ref[...] = ids.reshape(1, 1, TILE)

    enc_b = enc.astype(jnp.bfloat16)
    ehi = emb.astype(jnp.bfloat16)
    elo = (emb - ehi.astype(jnp.float32)).astype(jnp.bfloat16)
    quant = (jax.lax.dot_general(enc_b, ehi, (((1,), (1,)), ((), ())),
                                 preferred_element_type=jnp.float32)
             + jax.lax.dot_general(enc_b, elo, (((1,), (1,)), ((), ())),
                                   preferred_element_type=jnp.float32))
    quant_ref[...] = quant

    part_counts = jnp.sum(enc, axis=0, keepdims=True)
    diff = x - quant
    part_loss = jnp.sum(diff * diff)

    @pl.when(i == 0)
    def _init():
        counts_ref[...] = part_counts
        loss_acc_ref[0, 0] = part_loss

    @pl.when(i > 0)
    def _acc():
        counts_ref[...] += part_counts
        loss_acc_ref[0, 0] += part_loss

    @pl.when(i == STEPS - 1)
    def _fin():
        loss_val = BETA * loss_acc_ref[0, 0] / (N * D)
        loss_ref[...] = jnp.full((1, 1), loss_val, jnp.float32)
        p = counts_ref[...] * (1.0 / N)
        ent = jnp.sum(p * jnp.log(p + 1e-10))
        perp_ref[...] = jnp.full((1, 1), jnp.exp(-ent), jnp.float32)


def kernel(x, embedding):
    B, _, H, W = x.shape
    xp = jnp.transpose(x, (0, 2, 3, 1))
    x_flat = xp.reshape(-1, D)
    xsq = jnp.sum(x_flat ** 2, axis=1, keepdims=True)
    esq = jnp.sum(embedding ** 2, axis=0, keepdims=True)

    grid = (STEPS,)
    dist_out, enc_out, quant_out, ids_out, loss_out, perp_out = pl.pallas_call(
        _vq_kernel,
        grid=grid,
        in_specs=[
            pl.BlockSpec((TILE, D), lambda i: (i, 0)),
            pl.BlockSpec((TILE, 1), lambda i: (i, 0)),
            pl.BlockSpec((1, K), lambda i: (0, 0)),
            pl.BlockSpec((D, K), lambda i: (0, 0)),
        ],
        out_specs=[
            pl.BlockSpec((TILE, K), lambda i: (i, 0)),
            pl.BlockSpec((TILE, K), lambda i: (i, 0)),
            pl.BlockSpec((TILE, D), lambda i: (i, 0)),
            pl.BlockSpec((1, 1, TILE), lambda i: (i, 0, 0)),
            pl.BlockSpec((1, 1), lambda i: (0, 0)),
            pl.BlockSpec((1, 1), lambda i: (0, 0)),
        ],
        out_shape=[
            jax.ShapeDtypeStruct((N, K), jnp.float32),
            jax.ShapeDtypeStruct((N, K), jnp.float32),
            jax.ShapeDtypeStruct((N, D), jnp.float32),
            jax.ShapeDtypeStruct((STEPS, 1, TILE), jnp.int32),
            jax.ShapeDtypeStruct((1, 1), jnp.float32),
            jax.ShapeDtypeStruct((1, 1), jnp.float32),
        ],
        scratch_shapes=[
            pltpu.VMEM((1, K), jnp.float32),
            pltpu.SMEM((1, 1), jnp.float32),
        ],
        compiler_params=pltpu.CompilerParams(
            dimension_semantics=("arbitrary",),
        ),
    )(x_flat, xsq, esq, embedding)

    out = jnp.transpose(quant_out.reshape(B, H, W, D), (0, 3, 1, 2))
    loss = loss_out[0, 0]
    perplexity = perp_out[0, 0]
    ids_grid = ids_out.reshape(B, H, W)
    return (out, loss, perplexity, enc_out, ids_grid, dist_out)
